# Initial kernel scaffold; baseline (speedup 1.0000x reference)
import jax
import jax.numpy as jnp
from jax.experimental import pallas as pl


def kernel(x_nchw, dw_weight, pw_weight):
    raise NotImplementedError("write your pallas kernel here")


# speedup vs baseline: 1.3705x; 1.3705x over previous
import functools

import jax
import jax.numpy as jnp
from jax.experimental import pallas as pl
from jax.experimental.pallas import tpu as pltpu


def _vmem_limit_bytes():
    cap = 64 * 1024 * 1024
    return int(min((cap * 3) // 4, 100 * 1024 * 1024))


def _dwsep_body(x_ref, taps_ref, mask_ref, pw_ref, o_ref, *, width, ksize, pad):
    hw = x_ref.shape[-1]
    x = x_ref[0].astype(jnp.bfloat16)
    taps = taps_ref[...]
    mask = mask_ref[...]

    acc = jnp.zeros(x.shape, jnp.float32)
    for kh in range(ksize):
        for kw in range(ksize):
            t = kh * ksize + kw
            off = (kh - pad) * width + (kw - pad)
            shifted = x if off == 0 else pltpu.roll(x, (-off) % hw, axis=1)
            term = (shifted * mask[t][None, :]) * taps[t][:, None]
            acc = acc + term.astype(jnp.float32)

    y = jnp.dot(pw_ref[...], acc.astype(jnp.bfloat16),
                preferred_element_type=jnp.float32)
    o_ref[0] = y.astype(o_ref.dtype)


def kernel(x_nchw, dw_weight, pw_weight):
    n, c, h, w = x_nchw.shape
    k = dw_weight.shape[-1]
    o = pw_weight.shape[0]
    pad = (k - 1) // 2
    hw = h * w

    x3 = x_nchw.reshape(n, c, hw)

    taps = jnp.transpose(dw_weight[:, 0], (1, 2, 0)).reshape(k * k, c)
    taps = taps.astype(jnp.bfloat16)

    hh = jnp.arange(h)[:, None]
    ww = jnp.arange(w)[None, :]
    masks = []
    for kh in range(k):
        for kw in range(k):
            dh, dw_ = kh - pad, kw - pad
            valid = ((hh + dh >= 0) & (hh + dh < h) &
                     (ww + dw_ >= 0) & (ww + dw_ < w))
            masks.append(valid.reshape(hw))
    mask = jnp.stack(masks).astype(jnp.bfloat16)

    pw_mat = pw_weight[:, :, 0, 0].astype(jnp.bfloat16)

    body = functools.partial(_dwsep_body, width=w, ksize=k, pad=pad)

    out3 = pl.pallas_call(
        body,
        out_shape=jax.ShapeDtypeStruct((n, o, hw), x_nchw.dtype),
        grid=(n,),
        in_specs=[
            pl.BlockSpec((1, c, hw), lambda b: (b, 0, 0)),
            pl.BlockSpec((k * k, c), lambda b: (0, 0)),
            pl.BlockSpec((k * k, hw), lambda b: (0, 0)),
            pl.BlockSpec((o, c), lambda b: (0, 0)),
        ],
        out_specs=pl.BlockSpec((1, o, hw), lambda b: (b, 0, 0)),
        compiler_params=pltpu.CompilerParams(
            dimension_semantics=("parallel",),
            vmem_limit_bytes=_vmem_limit_bytes(),
        ),
    )(x3, taps, mask, pw_mat)

    return out3.reshape(n, o, h, w)
